# baseline (speedup 1.0000x reference)
import functools

import jax
import jax.numpy as jnp
from jax import lax
from jax.experimental import pallas as pl
from jax.experimental.pallas import tpu as pltpu
from jax.experimental.pallas import tpu_sc as plsc

S = 200
B = 4096
MAX_SEQ = 200
D = 64
SB = S * B
NC = 2
NS = 16
NW = NC * NS
RW = SB // NW
C = 512
GSUB = 128
NCHUNK = RW // C
G = C // 16
EPS = 1e-5 / 64.0


def _rsqrt(v):
    i = lax.bitcast_convert_type(v, jnp.int32)
    i = jnp.int32(0x5F3759DF) - (i >> 1)
    y = lax.bitcast_convert_type(i, jnp.float32)
    h = v * jnp.float32(0.5)
    for _ in range(3):
        y = y * (jnp.float32(1.5) - h * y * y)
    return y


def _body(seq_hbm, pid_hbm, item_hbm, pos_hbm, w_hbm, b_hbm, out_hbm,
          idx_a, idx_p, buf_rows, buf_x, pos_v, w_v, b_v, sem):
    wid = lax.axis_index("c") * NS + lax.axis_index("s")
    base0 = wid * RW

    pltpu.sync_copy(pos_hbm, pos_v)
    pltpu.sync_copy(w_hbm, w_v)
    pltpu.sync_copy(b_hbm, b_v)

    def _scale(i, carry):
        sl = pl.ds(i * 16, 16)
        pos_v[sl] = pos_v[sl] * jnp.float32(0.125)
        return carry
    lax.fori_loop(0, (MAX_SEQ * D) // 16, _scale, 0)

    iota16 = lax.iota(jnp.int32, 16)
    w_regs = [w_v[pl.ds(k * 16, 16)] for k in range(4)]
    b_regs = [b_v[pl.ds(k * 16, 16)] for k in range(4)]
    zero_f = jnp.zeros((16,), jnp.float32)
    col0 = jnp.zeros((16,), jnp.int32)

    def _chunk(c, carry):
        base = base0 + c * C
        pltpu.sync_copy(seq_hbm.at[pl.ds(base, C)], idx_a)
        pltpu.sync_copy(pid_hbm.at[pl.ds(base, C)], idx_p)
        cps = [
            pltpu.async_copy(
                item_hbm.at[idx_a.at[pl.ds(j * GSUB, GSUB)]],
                buf_rows.at[pl.ds(j * GSUB, GSUB)],
                sem,
            )
            for j in range(C // GSUB)
        ]
        for cp in cps:
            cp.wait()

        def _group(g, carry):
            rvec = g * 16 + iota16
            pld = idx_p[pl.ds(g * 16, 16)]
            ip0 = pld * D
            ix0 = rvec * D

            def _p1(d, st):
                col, ia, ip, s1, s2 = st
                a = plsc.load_gather(buf_rows, [rvec, col])
                p = plsc.load_gather(pos_v, [ip])
                x = a + p
                plsc.store_scatter(buf_x, [ia], x)
                return (col + 1, ia + 1, ip + 1, s1 + x, s2 + x * x)

            _, _, _, s1, s2 = lax.fori_loop(
                0, D, _p1, (col0, ix0, ip0, zero_f, zero_f))
            m = s1 * jnp.float32(1.0 / D)
            var = s2 * jnp.float32(1.0 / D) - m * m + jnp.float32(EPS)
            s = _rsqrt(var)
            u = m * s
            goff = g * (16 * D)
            for r in range(16):
                sr = s[r]
                ur = u[r]
                for k in range(4):
                    sl = pl.ds(goff + r * D + k * 16, 16)
                    x = buf_x[sl]
                    buf_x[sl] = (x * sr - ur) * w_regs[k] + b_regs[k]
            return carry

        lax.fori_loop(0, G, _group, 0)
        pltpu.sync_copy(buf_x, out_hbm.at[pl.ds(base * D, C * D)])
        return carry

    lax.fori_loop(0, NCHUNK, _chunk, 0)


@jax.jit
def _emb(seq_flat, pid_flat, item_table, pos_flat, ln_weight, ln_bias):
    mesh = plsc.VectorSubcoreMesh(core_axis_name="c", subcore_axis_name="s")
    f = functools.partial(
        pl.kernel,
        out_type=jax.ShapeDtypeStruct((SB * D,), jnp.float32),
        mesh=mesh,
        scratch_types=[
            pltpu.VMEM((C,), jnp.int32),
            pltpu.VMEM((C,), jnp.int32),
            pltpu.VMEM((C, D), jnp.float32),
            pltpu.VMEM((C * D,), jnp.float32),
            pltpu.VMEM((MAX_SEQ * D,), jnp.float32),
            pltpu.VMEM((D,), jnp.float32),
            pltpu.VMEM((D,), jnp.float32),
            pltpu.SemaphoreType.DMA,
        ],
        compiler_params=pltpu.CompilerParams(
            needs_layout_passes=False, use_tc_tiling_on_sc=False),
    )(_body)
    return f(seq_flat, pid_flat, item_table, pos_flat, ln_weight, ln_bias)


def kernel(input_sequence, position_ids, item_table, pos_table, ln_weight, ln_bias):
    seq_flat = input_sequence.reshape(SB)
    pid_flat = position_ids.reshape(SB)
    pos_flat = pos_table.reshape(MAX_SEQ * D)
    out = _emb(seq_flat, pid_flat, item_table, pos_flat, ln_weight, ln_bias)
    return out.reshape(S, B, D)
